# baseline (speedup 1.0000x reference)
import functools

import jax
import jax.numpy as jnp
from jax import lax
from jax.experimental import pallas as pl
from jax.experimental.pallas import tpu as pltpu
from jax.experimental.pallas import tpu_sc as plsc

NUM_EMB = 100000
DIM = 128
BATCH = 4096
HIST = 200
B_TOTAL = BATCH * HIST
NC = 2
NS = 16
NW = NC * NS
B_PER_W = B_TOTAL // NW
CHUNK = 160
N_CHUNKS = B_PER_W // CHUNK


NBUF = 4
LOOKAHEAD = 2
N_ROUNDS = N_CHUNKS // NBUF


def _emb_body(idx_hbm, table_hbm, out_hbm, idx_v, rows_v, *sems):
    gsems = sems[:NBUF]
    osems = sems[NBUF:]
    wid = lax.axis_index("s") * NC + lax.axis_index("c")
    base = wid * B_PER_W
    pltpu.sync_copy(idx_hbm.at[pl.ds(base, B_PER_W)], idx_v)

    def g_copy(ch, b):
        return pltpu.make_async_copy(
            table_hbm.at[idx_v.at[pl.ds(ch * CHUNK, CHUNK)]],
            rows_v.at[b],
            gsems[b],
        )

    def o_copy(ch, b):
        return pltpu.make_async_copy(
            rows_v.at[b], out_hbm.at[pl.ds(base + ch * CHUNK, CHUNK)], osems[b]
        )

    L = LOOKAHEAD

    for ch in range(L):
        g_copy(ch, ch % NBUF).start()

    for ch in range(NBUF):
        ahead = ch + L
        if ahead >= NBUF:
            o_copy(ahead - NBUF, ahead % NBUF).wait()
        g_copy(ahead, ahead % NBUF).start()
        g_copy(ch, ch % NBUF).wait()
        o_copy(ch, ch % NBUF).start()

    def main_body(r, carry):
        ch0 = r * NBUF
        for b in range(NBUF):
            ch = ch0 + b
            ahead = ch + L
            ba = (b + L) % NBUF
            o_copy(ahead - NBUF, ba).wait()
            g_copy(ahead, ba).start()
            g_copy(ch, b).wait()
            o_copy(ch, b).start()
        return carry

    lax.fori_loop(1, N_ROUNDS - 1, main_body, 0)

    ch0 = (N_ROUNDS - 1) * NBUF
    for b in range(NBUF):
        ch = ch0 + b
        ahead = ch + L
        if ahead < N_CHUNKS:
            ba = (b + L) % NBUF
            o_copy(ahead - NBUF, ba).wait()
            g_copy(ahead, ba).start()
        g_copy(ch, b).wait()
        o_copy(ch, b).start()

    for b in range(NBUF):
        o_copy(ch0 + b, b).wait()


@functools.partial(jax.jit)
def _embedding_lookup(flat_idx, table):
    mesh = plsc.VectorSubcoreMesh(core_axis_name="c", subcore_axis_name="s")
    k = functools.partial(
        pl.kernel,
        mesh=mesh,
        out_type=jax.ShapeDtypeStruct((B_TOTAL, DIM), jnp.float32),
        scratch_types=[
            pltpu.VMEM((B_PER_W,), jnp.int32),
            pltpu.VMEM((NBUF, CHUNK, DIM), jnp.float32),
        ]
        + [pltpu.SemaphoreType.DMA] * (2 * NBUF),
    )(_emb_body)
    return k(flat_idx, table)


def kernel(token_ids, embedding_matrix):
    flat_idx = token_ids.reshape(-1)
    out = _embedding_lookup(flat_idx, embedding_matrix)
    return out.reshape(BATCH, HIST, DIM)


# speedup vs baseline: 1.0032x; 1.0032x over previous
import functools

import jax
import jax.numpy as jnp
from jax import lax
from jax.experimental import pallas as pl
from jax.experimental.pallas import tpu as pltpu
from jax.experimental.pallas import tpu_sc as plsc

NUM_EMB = 100000
DIM = 128
BATCH = 4096
HIST = 200
B_TOTAL = BATCH * HIST
NC = 2
NS = 16
NW = NC * NS
B_PER_W = B_TOTAL // NW
CHUNK = 80
N_CHUNKS = B_PER_W // CHUNK


NBUF = 10
LOOKAHEAD = 4
N_ROUNDS = N_CHUNKS // NBUF


def _emb_body(idx_hbm, table_hbm, out_hbm, idx_v, rows_v, *sems):
    gsems = sems[:NBUF]
    osems = sems[NBUF:]
    wid = lax.axis_index("s") * NC + lax.axis_index("c")
    base = wid * B_PER_W
    pltpu.sync_copy(idx_hbm.at[pl.ds(base, B_PER_W)], idx_v)

    def g_copy(ch, b):
        return pltpu.make_async_copy(
            table_hbm.at[idx_v.at[pl.ds(ch * CHUNK, CHUNK)]],
            rows_v.at[b],
            gsems[b],
        )

    def o_copy(ch, b):
        return pltpu.make_async_copy(
            rows_v.at[b], out_hbm.at[pl.ds(base + ch * CHUNK, CHUNK)], osems[b]
        )

    L = LOOKAHEAD

    for ch in range(L):
        g_copy(ch, ch % NBUF).start()

    for ch in range(NBUF):
        ahead = ch + L
        if ahead >= NBUF:
            o_copy(ahead - NBUF, ahead % NBUF).wait()
        g_copy(ahead, ahead % NBUF).start()
        g_copy(ch, ch % NBUF).wait()
        o_copy(ch, ch % NBUF).start()

    def main_body(r, carry):
        ch0 = r * NBUF
        for b in range(NBUF):
            ch = ch0 + b
            ahead = ch + L
            ba = (b + L) % NBUF
            o_copy(ahead - NBUF, ba).wait()
            g_copy(ahead, ba).start()
            g_copy(ch, b).wait()
            o_copy(ch, b).start()
        return carry

    lax.fori_loop(1, N_ROUNDS - 1, main_body, 0)

    ch0 = (N_ROUNDS - 1) * NBUF
    for b in range(NBUF):
        ch = ch0 + b
        ahead = ch + L
        if ahead < N_CHUNKS:
            ba = (b + L) % NBUF
            o_copy(ahead - NBUF, ba).wait()
            g_copy(ahead, ba).start()
        g_copy(ch, b).wait()
        o_copy(ch, b).start()

    for b in range(NBUF):
        o_copy(ch0 + b, b).wait()


@functools.partial(jax.jit)
def _embedding_lookup(flat_idx, table):
    mesh = plsc.VectorSubcoreMesh(core_axis_name="c", subcore_axis_name="s")
    k = functools.partial(
        pl.kernel,
        mesh=mesh,
        out_type=jax.ShapeDtypeStruct((B_TOTAL, DIM), jnp.float32),
        scratch_types=[
            pltpu.VMEM((B_PER_W,), jnp.int32),
            pltpu.VMEM((NBUF, CHUNK, DIM), jnp.float32),
        ]
        + [pltpu.SemaphoreType.DMA] * (2 * NBUF),
    )(_emb_body)
    return k(flat_idx, table)


def kernel(token_ids, embedding_matrix):
    flat_idx = token_ids.reshape(-1)
    out = _embedding_lookup(flat_idx, embedding_matrix)
    return out.reshape(BATCH, HIST, DIM)
